# baseline (speedup 1.0000x reference)
import functools

import jax
import jax.numpy as jnp
from jax import lax
from jax.experimental import pallas as pl
from jax.experimental.pallas import tpu as pltpu
from jax.experimental.pallas import tpu_sc as plsc

_BATCH = 16384
_NUM_TOKENS = 32
_CODEBOOK_SIZE = 64
_HIDDEN = 128


@functools.lru_cache(maxsize=None)
def _build_sc_counts():
    info = plsc.get_sparse_core_info()
    nc, ns, lanes = info.num_cores, info.num_subcores, info.num_lanes
    nw = nc * ns
    rpw = _BATCH // nw

    mesh = plsc.VectorSubcoreMesh(core_axis_name="c", subcore_axis_name="s")

    @functools.partial(
        pl.kernel,
        out_type=jax.ShapeDtypeStruct((_BATCH, _CODEBOOK_SIZE), jnp.float32),
        mesh=mesh,
        scratch_types=[
            pltpu.VMEM((rpw, _NUM_TOKENS), jnp.int32),
            pltpu.VMEM((rpw, _CODEBOOK_SIZE), jnp.float32),
            pltpu.SemaphoreType.DMA,
            pltpu.SemaphoreType.DMA,
        ],
        compiler_params=pltpu.CompilerParams(
            needs_layout_passes=False, use_tc_tiling_on_sc=True
        ),
    )
    def sc_counts(idx_hbm, cnt_hbm, idx_v, cnt_v, sem_in, sem_out):
        wid = lax.axis_index("s") * nc + lax.axis_index("c")
        base = wid * rpw
        half = rpw // 2
        pltpu.sync_copy(idx_hbm.at[pl.ds(base, half)], idx_v.at[pl.ds(0, half)])
        in1 = pltpu.async_copy(
            idx_hbm.at[pl.ds(base + half, half)],
            idx_v.at[pl.ds(half, half)],
            sem_in,
        )

        zero = jnp.zeros((lanes,), jnp.float32)
        ones = jnp.ones((lanes,), jnp.float32)

        def hist_row(r):
            rows = jnp.full((lanes,), r, jnp.int32)
            for c in range(_CODEBOOK_SIZE // lanes):
                cnt_v[r, pl.ds(c * lanes, lanes)] = zero
            for t0 in range(_NUM_TOKENS // lanes):
                iv = idx_v[r, pl.ds(t0 * lanes, lanes)]
                plsc.addupdate_scatter(cnt_v, [rows, iv], ones)

        plsc.parallel_loop(0, half, unroll=4)(hist_row)
        out0 = pltpu.async_copy(
            cnt_v.at[pl.ds(0, half)], cnt_hbm.at[pl.ds(base, half)], sem_out
        )
        in1.wait()
        plsc.parallel_loop(half, rpw, unroll=4)(hist_row)
        out0.wait()
        pltpu.sync_copy(
            cnt_v.at[pl.ds(half, half)], cnt_hbm.at[pl.ds(base + half, half)]
        )

    return sc_counts


def _mm_body(cnt_ref, cb_ref, out_ref):
    out_ref[...] = lax.dot_general(
        cnt_ref[...],
        cb_ref[...],
        (((1,), (0,)), ((), ())),
        preferred_element_type=jnp.float32,
        precision=lax.Precision.DEFAULT,
    )


def kernel(indices, codebook):
    counts = _build_sc_counts()(indices)
    bm = 8192
    out = pl.pallas_call(
        _mm_body,
        grid=(_BATCH // bm,),
        in_specs=[
            pl.BlockSpec((bm, _CODEBOOK_SIZE), lambda i: (i, 0)),
            pl.BlockSpec((_CODEBOOK_SIZE, _HIDDEN), lambda i: (0, 0)),
        ],
        out_specs=pl.BlockSpec((bm, _HIDDEN), lambda i: (i, 0)),
        out_shape=jax.ShapeDtypeStruct((_BATCH, _HIDDEN), jnp.float32),
    )(counts, codebook)
    return out


# speedup vs baseline: 1.0019x; 1.0019x over previous
import functools

import jax
import jax.numpy as jnp
from jax import lax
from jax.experimental import pallas as pl
from jax.experimental.pallas import tpu as pltpu
from jax.experimental.pallas import tpu_sc as plsc

_BATCH = 16384
_NUM_TOKENS = 32
_CODEBOOK_SIZE = 64
_HIDDEN = 128


@functools.lru_cache(maxsize=None)
def _build_sc_counts():
    info = plsc.get_sparse_core_info()
    nc, ns, lanes = info.num_cores, info.num_subcores, info.num_lanes
    nw = nc * ns
    rpw = _BATCH // nw

    mesh = plsc.VectorSubcoreMesh(core_axis_name="c", subcore_axis_name="s")

    @functools.partial(
        pl.kernel,
        out_type=jax.ShapeDtypeStruct((_BATCH, _CODEBOOK_SIZE), jnp.float32),
        mesh=mesh,
        scratch_types=[
            pltpu.VMEM((rpw, _NUM_TOKENS), jnp.int32),
            pltpu.VMEM((rpw, _CODEBOOK_SIZE), jnp.float32),
            pltpu.SemaphoreType.DMA,
            pltpu.SemaphoreType.DMA,
        ],
        compiler_params=pltpu.CompilerParams(
            needs_layout_passes=False, use_tc_tiling_on_sc=True
        ),
    )
    def sc_counts(idx_hbm, cnt_hbm, idx_v, cnt_v, sem_in, sem_out):
        wid = lax.axis_index("s") * nc + lax.axis_index("c")
        base = wid * rpw
        nparts = 4
        part = rpw // nparts
        pltpu.sync_copy(idx_hbm.at[pl.ds(base, part)], idx_v.at[pl.ds(0, part)])
        ins = [
            pltpu.async_copy(
                idx_hbm.at[pl.ds(base + p * part, part)],
                idx_v.at[pl.ds(p * part, part)],
                sem_in,
            )
            for p in range(1, nparts)
        ]

        zero = jnp.zeros((lanes,), jnp.float32)
        ones = jnp.ones((lanes,), jnp.float32)

        def hist_row(r):
            rows = jnp.full((lanes,), r, jnp.int32)
            for c in range(_CODEBOOK_SIZE // lanes):
                cnt_v[r, pl.ds(c * lanes, lanes)] = zero
            for t0 in range(_NUM_TOKENS // lanes):
                iv = idx_v[r, pl.ds(t0 * lanes, lanes)]
                plsc.addupdate_scatter(cnt_v, [rows, iv], ones)

        outs = []
        for p in range(nparts):
            plsc.parallel_loop(p * part, (p + 1) * part, unroll=4)(hist_row)
            outs.append(
                pltpu.async_copy(
                    cnt_v.at[pl.ds(p * part, part)],
                    cnt_hbm.at[pl.ds(base + p * part, part)],
                    sem_out,
                )
            )
            if p + 1 < nparts:
                ins[p].wait()
        for o in outs:
            o.wait()

    return sc_counts


def _mm_body(cnt_ref, cb_ref, out_ref):
    out_ref[...] = lax.dot_general(
        cnt_ref[...],
        cb_ref[...],
        (((1,), (0,)), ((), ())),
        preferred_element_type=jnp.float32,
        precision=lax.Precision.DEFAULT,
    )


def kernel(indices, codebook):
    counts = _build_sc_counts()(indices)
    bm = 8192
    out = pl.pallas_call(
        _mm_body,
        grid=(_BATCH // bm,),
        in_specs=[
            pl.BlockSpec((bm, _CODEBOOK_SIZE), lambda i: (i, 0)),
            pl.BlockSpec((_CODEBOOK_SIZE, _HIDDEN), lambda i: (0, 0)),
        ],
        out_specs=pl.BlockSpec((bm, _HIDDEN), lambda i: (i, 0)),
        out_shape=jax.ShapeDtypeStruct((_BATCH, _HIDDEN), jnp.float32),
    )(counts, codebook)
    return out
